# baseline (speedup 1.0000x reference)
import jax
import jax.numpy as jnp
from jax.experimental import pallas as pl
from jax.experimental.pallas import tpu as pltpu

H0, W0, C0 = 32, 32, 3
KH = 5
OC1, OC2 = 6, 16
WP1, WP2 = 14, 5
FC1_OUT, FC2_OUT, FC3_OUT = 120, 84, 10
L1 = 128
L2 = 128
B = 512


def _fused_kernel(x_ref, w1_ref, b1_ref, w2_ref, b2_ref, wf1_ref, bf1_ref,
                  wf2_ref, bf2_ref, wf3_ref, bf3_ref, o_ref):
    f32 = jnp.float32
    bf16 = jnp.bfloat16
    R1 = B * H0

    xcat = x_ref[...].reshape(R1, W0 * C0).astype(bf16)
    taps = [xcat] + [pltpu.roll(xcat, R1 - t, 0) for t in range(1, KH)]
    lhs1 = jnp.concatenate(taps, axis=1)
    S = jnp.dot(lhs1, w1_ref[...], preferred_element_type=f32)
    ym = jnp.maximum(S[:, :L1], S[:, L1:])
    f1 = ym.reshape(R1 // 2, 2 * L1)
    p1 = jnp.maximum(f1[:, :L1], f1[:, L1:])
    a1 = jnp.maximum(p1 + b1_ref[...], 0.0).astype(bf16)

    R2 = R1 // 2
    a1d = a1[:, :WP1 * OC1]
    taps2 = [a1d] + [pltpu.roll(a1d, R2 - t, 0) for t in range(1, KH)]
    lhs2 = jnp.concatenate(taps2, axis=1)
    S2 = jnp.dot(lhs2, w2_ref[...], preferred_element_type=f32)
    ym2 = jnp.maximum(S2[:, :L2], S2[:, L2:])
    f2 = ym2.reshape(R2 // 2, 2 * L2)
    p2 = jnp.maximum(f2[:, :L2], f2[:, L2:])
    a2 = jnp.maximum(p2 + b2_ref[...], 0.0).astype(bf16)

    A2 = a2.reshape(B, 8 * L2)
    h1 = jnp.dot(A2, wf1_ref[...], preferred_element_type=f32) + bf1_ref[...]
    h1 = jnp.maximum(h1, 0.0).astype(bf16)
    h2 = jnp.dot(h1, wf2_ref[...], preferred_element_type=f32) + bf2_ref[...]
    h2 = jnp.maximum(h2, 0.0).astype(bf16)
    o_ref[...] = jnp.dot(h2, wf3_ref[...], preferred_element_type=f32) + bf3_ref[...]


@jax.jit
def kernel(x_nchw, w1, b1, w2, b2, wf1, bf1, wf2, bf2, wf3, bf3):
    n = x_nchw.shape[0]
    npad = -(-n // B) * B
    x = jnp.transpose(x_nchw, (0, 2, 3, 1)).reshape(n, H0, W0 * C0)
    if npad != n:
        x = jnp.pad(x, ((0, npad - n), (0, 0), (0, 0)))

    NB1 = WP1 * OC1
    NB2 = WP2 * OC2
    w1p = jnp.pad(w1, ((0, 0), (0, 0), (0, 0), (0, L1 - NB1)))
    W1big = jnp.concatenate([w1p[:, 0], w1p[:, 1]], axis=2).reshape(KH * C0 * W0, 2 * L1)
    w2p = jnp.pad(w2, ((0, 0), (0, 0), (0, 0), (0, L2 - NB2)))
    W2big = jnp.concatenate([w2p[:, 0], w2p[:, 1]], axis=2).reshape(KH * NB1, 2 * L2)
    wf1p = jnp.pad(wf1, ((0, 3), (0, L2 - NB2), (0, 0))).reshape(8 * L2, FC1_OUT)
    b1p = jnp.pad(b1, ((0, 0), (0, L1 - NB1)))
    b2p = jnp.pad(b2, ((0, 0), (0, L2 - NB2)))

    grid = (npad // B,)

    def full(nd):
        return lambda i: (0,) * nd

    out = pl.pallas_call(
        _fused_kernel,
        out_shape=jax.ShapeDtypeStruct((npad, FC3_OUT), jnp.float32),
        grid=grid,
        in_specs=[
            pl.BlockSpec((B, H0, W0 * C0), lambda i: (i, 0, 0)),
            pl.BlockSpec((KH * C0 * W0, 2 * L1), full(2)),
            pl.BlockSpec((1, L1), full(2)),
            pl.BlockSpec((KH * NB1, 2 * L2), full(2)),
            pl.BlockSpec((1, L2), full(2)),
            pl.BlockSpec((8 * L2, FC1_OUT), full(2)),
            pl.BlockSpec((1, FC1_OUT), full(2)),
            pl.BlockSpec((FC1_OUT, FC2_OUT), full(2)),
            pl.BlockSpec((1, FC2_OUT), full(2)),
            pl.BlockSpec((FC2_OUT, FC3_OUT), full(2)),
            pl.BlockSpec((1, FC3_OUT), full(2)),
        ],
        out_specs=pl.BlockSpec((B, FC3_OUT), lambda i: (i, 0)),
        compiler_params=pltpu.CompilerParams(dimension_semantics=("parallel",)),
    )(x, W1big, b1p, W2big, b2p, wf1p, bf1, wf2, bf2, wf3, bf3)
    return out[:n]


# speedup vs baseline: 1.2432x; 1.2432x over previous
import jax
import jax.numpy as jnp
from jax.experimental import pallas as pl
from jax.experimental.pallas import tpu as pltpu

H0, W0, C0 = 32, 32, 3
KH = 5
OC1, OC2 = 6, 16
WP1, WP2 = 14, 5
FC1_OUT, FC2_OUT, FC3_OUT = 120, 84, 10
L1 = 128
L2 = 128
B = 512


def _fused_kernel(x_ref, w1_ref, b1_ref, w2_ref, b2_ref, wf1_ref, bf1_ref,
                  wf2_ref, bf2_ref, wf3_ref, bf3_ref, o_ref):
    f32 = jnp.float32
    bf16 = jnp.bfloat16
    R1 = B * H0

    xcat = x_ref[...].reshape(R1, W0 * C0)
    taps = [xcat] + [pltpu.roll(xcat, R1 - t, 0) for t in range(1, KH)]
    lhs1 = jnp.concatenate(taps, axis=1)
    S = jnp.dot(lhs1, w1_ref[...], preferred_element_type=f32)
    ym = jnp.maximum(S[:, :L1], S[:, L1:])
    f1 = ym.reshape(R1 // 2, 2 * L1)
    p1 = jnp.maximum(f1[:, :L1], f1[:, L1:])
    a1 = jnp.maximum(p1 + b1_ref[...], 0.0).astype(bf16)

    R2 = R1 // 2
    a1d = a1[:, :WP1 * OC1]
    taps2 = [a1d] + [pltpu.roll(a1d, R2 - t, 0) for t in range(1, KH)]
    lhs2 = jnp.concatenate(taps2, axis=1)
    S2 = jnp.dot(lhs2, w2_ref[...], preferred_element_type=f32)
    ym2 = jnp.maximum(S2[:, :L2], S2[:, L2:])
    f2 = ym2.reshape(R2 // 2, 2 * L2)
    p2 = jnp.maximum(f2[:, :L2], f2[:, L2:])
    a2 = jnp.maximum(p2 + b2_ref[...], 0.0).astype(bf16)

    A2 = a2.reshape(B, 8 * L2)
    h1 = jnp.dot(A2, wf1_ref[...], preferred_element_type=f32) + bf1_ref[...]
    h1 = jnp.maximum(h1, 0.0).astype(bf16)
    h2 = jnp.dot(h1, wf2_ref[...], preferred_element_type=f32) + bf2_ref[...]
    h2 = jnp.maximum(h2, 0.0).astype(bf16)
    o_ref[...] = jnp.dot(h2, wf3_ref[...], preferred_element_type=f32) + bf3_ref[...]


@jax.jit
def kernel(x_nchw, w1, b1, w2, b2, wf1, bf1, wf2, bf2, wf3, bf3):
    n = x_nchw.shape[0]
    npad = -(-n // B) * B
    x = jnp.swapaxes(x_nchw.astype(jnp.bfloat16), 1, 2)
    x = x.reshape(n, H0, W0 * C0)
    if npad != n:
        x = jnp.pad(x, ((0, npad - n), (0, 0), (0, 0)))

    NB1 = WP1 * OC1
    NB2 = WP2 * OC2
    w1r = w1.reshape(KH, 2, W0, C0, NB1).transpose(0, 1, 3, 2, 4)
    w1r = w1r.reshape(KH, 2, C0 * W0, NB1)
    w1p = jnp.pad(w1r, ((0, 0), (0, 0), (0, 0), (0, L1 - NB1)))
    W1big = jnp.concatenate([w1p[:, 0], w1p[:, 1]], axis=2).reshape(KH * C0 * W0, 2 * L1)
    w2p = jnp.pad(w2, ((0, 0), (0, 0), (0, 0), (0, L2 - NB2)))
    W2big = jnp.concatenate([w2p[:, 0], w2p[:, 1]], axis=2).reshape(KH * NB1, 2 * L2)
    wf1p = jnp.pad(wf1, ((0, 3), (0, L2 - NB2), (0, 0))).reshape(8 * L2, FC1_OUT)
    b1p = jnp.pad(b1, ((0, 0), (0, L1 - NB1)))
    b2p = jnp.pad(b2, ((0, 0), (0, L2 - NB2)))

    grid = (npad // B,)

    def full(nd):
        return lambda i: (0,) * nd

    out = pl.pallas_call(
        _fused_kernel,
        out_shape=jax.ShapeDtypeStruct((npad, FC3_OUT), jnp.float32),
        grid=grid,
        in_specs=[
            pl.BlockSpec((B, H0, W0 * C0), lambda i: (i, 0, 0)),
            pl.BlockSpec((KH * C0 * W0, 2 * L1), full(2)),
            pl.BlockSpec((1, L1), full(2)),
            pl.BlockSpec((KH * NB1, 2 * L2), full(2)),
            pl.BlockSpec((1, L2), full(2)),
            pl.BlockSpec((8 * L2, FC1_OUT), full(2)),
            pl.BlockSpec((1, FC1_OUT), full(2)),
            pl.BlockSpec((FC1_OUT, FC2_OUT), full(2)),
            pl.BlockSpec((1, FC2_OUT), full(2)),
            pl.BlockSpec((FC2_OUT, FC3_OUT), full(2)),
            pl.BlockSpec((1, FC3_OUT), full(2)),
        ],
        out_specs=pl.BlockSpec((B, FC3_OUT), lambda i: (i, 0)),
        compiler_params=pltpu.CompilerParams(dimension_semantics=("parallel",)),
    )(x, W1big, b1p, W2big, b2p, wf1p, bf1, wf2, bf2, wf3, bf3)
    return out[:n]
